# Initial kernel scaffold; baseline (speedup 1.0000x reference)
import jax
import jax.numpy as jnp
from jax.experimental import pallas as pl


def kernel(x, mbh, b0u0_conv1_w, b0u0_conv1_b, b0u0_conv2_w, b0u0_conv2_b, b0u0_bn1_g, b0u0_bn1_be, b0u0_bn1_m, b0u0_bn1_v, b0u0_bn2_g, b0u0_bn2_be, b0u0_bn2_m, b0u0_bn2_v, b0u0_conv3_w, b0u0_conv3_b, b0u1_conv1_w, b0u1_conv1_b, b0u1_conv2_w, b0u1_conv2_b, b0u1_bn1_g, b0u1_bn1_be, b0u1_bn1_m, b0u1_bn1_v, b0u1_bn2_g, b0u1_bn2_be, b0u1_bn2_m, b0u1_bn2_v, b1u0_conv1_w, b1u0_conv1_b, b1u0_conv2_w, b1u0_conv2_b, b1u0_bn1_g, b1u0_bn1_be, b1u0_bn1_m, b1u0_bn1_v, b1u0_bn2_g, b1u0_bn2_be, b1u0_bn2_m, b1u0_bn2_v, b1u0_conv3_w, b1u0_conv3_b, b1u1_conv1_w, b1u1_conv1_b, b1u1_conv2_w, b1u1_conv2_b, b1u1_bn1_g, b1u1_bn1_be, b1u1_bn1_m, b1u1_bn1_v, b1u1_bn2_g, b1u1_bn2_be, b1u1_bn2_m, b1u1_bn2_v, b2u0_conv1_w, b2u0_conv1_b, b2u0_conv2_w, b2u0_conv2_b, b2u0_bn1_g, b2u0_bn1_be, b2u0_bn1_m, b2u0_bn1_v, b2u0_bn2_g, b2u0_bn2_be, b2u0_bn2_m, b2u0_bn2_v, b2u0_conv3_w, b2u0_conv3_b, b2u1_conv1_w, b2u1_conv1_b, b2u1_conv2_w, b2u1_conv2_b, b2u1_bn1_g, b2u1_bn1_be, b2u1_bn1_m, b2u1_bn1_v, b2u1_bn2_g, b2u1_bn2_be, b2u1_bn2_m, b2u1_bn2_v, b3u0_conv1_w, b3u0_conv1_b, b3u0_conv2_w, b3u0_conv2_b, b3u0_bn1_g, b3u0_bn1_be, b3u0_bn1_m, b3u0_bn1_v, b3u0_bn2_g, b3u0_bn2_be, b3u0_bn2_m, b3u0_bn2_v, b3u0_conv3_w, b3u0_conv3_b, b3u1_conv1_w, b3u1_conv1_b, b3u1_conv2_w, b3u1_conv2_b, b3u1_bn1_g, b3u1_bn1_be, b3u1_bn1_m, b3u1_bn1_v, b3u1_bn2_g, b3u1_bn2_be, b3u1_bn2_m, b3u1_bn2_v, w1_feat, w1_mbh, b1, w2, b2):
    raise NotImplementedError("write your pallas kernel here")


# speedup vs baseline: 3.9471x; 3.9471x over previous
import functools

import jax
import jax.numpy as jnp
from jax.experimental import pallas as pl
from jax.experimental.pallas import tpu as pltpu

_CDT = jnp.bfloat16
_BBLK = 8

_TAPS4 = tuple((di, dj) for di in range(2) for dj in range(2))
_TAPS9 = tuple((dh, dw) for dh in range(3) for dw in range(3))


def _unit_body(*refs, taps, Ho, Wo, C1, C2, use1x1, sc_h0, sc_lo, sc_hi,
               pad_out, fuse_head):
    it = iter(refs)
    x_ref, w1_ref, s1_ref, t1_ref, w2_ref, s2_ref, t2_ref = (
        next(it) for _ in range(7))
    if use1x1:
        ws_ref, tb_ref = next(it), next(it)
    if fuse_head:
        hw1_ref, hb1_ref, hw2_ref, hb2_ref = (next(it) for _ in range(4))
    out_ref = next(it)
    ypad_ref = next(it)

    x = x_ref[...]
    bblk = x.shape[0]
    m = bblk * Ho * Wo

    parts = [x[:, di:di + Ho, dj:dj + Wo, :] for (di, dj) in taps]
    p = jnp.concatenate(parts, axis=-1).reshape(m, -1)
    acc = jnp.dot(p, w1_ref[...], preferred_element_type=jnp.float32)
    y = jnp.maximum(acc * s1_ref[...] + t1_ref[...], 0.0)
    ypad_ref[:, 1:Ho + 1, 1:Wo + 1, :] = y.reshape(
        bblk, Ho, Wo, C1).astype(ypad_ref.dtype)
    zr = jnp.zeros((bblk, 1, Wo + 2, C1), ypad_ref.dtype)
    zc = jnp.zeros((bblk, Ho, 1, C1), ypad_ref.dtype)
    ypad_ref[:, 0:1, :, :] = zr
    ypad_ref[:, Ho + 1:Ho + 2, :, :] = zr
    ypad_ref[:, 1:Ho + 1, 0:1, :] = zc
    ypad_ref[:, 1:Ho + 1, Wo + 1:Wo + 2, :] = zc

    yp = ypad_ref[...]
    parts2 = [yp[:, dh:dh + Ho, dw:dw + Wo, :] for (dh, dw) in _TAPS9]
    p2 = jnp.concatenate(parts2, axis=-1).reshape(m, 9 * C1)
    acc2 = jnp.dot(p2, w2_ref[...], preferred_element_type=jnp.float32)
    acc2 = acc2 * s2_ref[...] + t2_ref[...]

    xs = x[:, sc_h0:sc_h0 + Ho, sc_h0:sc_h0 + Wo, sc_lo:sc_hi]
    if use1x1:
        sc = jnp.dot(xs.reshape(m, sc_hi - sc_lo), ws_ref[...],
                     preferred_element_type=jnp.float32) + tb_ref[...]
    else:
        sc = xs.reshape(m, C2).astype(jnp.float32)
    o = jnp.maximum(acc2 + sc, 0.0)

    if fuse_head:
        pooled = jnp.sum(o.reshape(bblk, Ho * Wo, C2), axis=1) * (
            1.0 / float(Ho * Wo))
        h = jnp.dot(pooled, hw1_ref[...],
                    preferred_element_type=jnp.float32) + hb1_ref[...]
        out_ref[...] = (jnp.sum(h * hw2_ref[...], axis=1, keepdims=True)
                        + hb2_ref[...]).astype(out_ref.dtype)
    elif pad_out:
        out_ref[:, 1:Ho + 1, 1:Wo + 1, :] = o.reshape(
            bblk, Ho, Wo, C2).astype(out_ref.dtype)
        zr2 = jnp.zeros((bblk, 1, Wo + 2, C2), out_ref.dtype)
        zc2 = jnp.zeros((bblk, Ho, 1, C2), out_ref.dtype)
        out_ref[:, 0:1, :, :] = zr2
        out_ref[:, Ho + 1:Ho + 2, :, :] = zr2
        out_ref[:, 1:Ho + 1, 0:1, :] = zc2
        out_ref[:, 1:Ho + 1, Wo + 1:Wo + 2, :] = zc2
    else:
        out_ref[...] = o.reshape(bblk, Ho, Wo, C2).astype(out_ref.dtype)


def _fold_bn(conv_bias, gamma, beta, mean, var, eps=1e-5):
    scale = gamma / jnp.sqrt(var + eps)
    shift = (conv_bias - mean) * scale + beta
    return scale, shift


def _row(v, n):
    return v.reshape(1, n).astype(jnp.float32)


def _res_unit(x, w1, s1, t1, w2, s2, t2, ws=None, tb=None, head=None, *,
              taps, Ho, Wo, sc_h0, sc_lo, sc_hi, pad_out, bblk=_BBLK):
    B, Hi, Wi, Ct = x.shape
    K1, C1 = w1.shape
    C2 = w2.shape[1]
    use1x1 = ws is not None
    fuse_head = head is not None
    grid = (B // bblk,)

    in_specs = [
        pl.BlockSpec((bblk, Hi, Wi, Ct), lambda i: (i, 0, 0, 0)),
        pl.BlockSpec((K1, C1), lambda i: (0, 0)),
        pl.BlockSpec((1, C1), lambda i: (0, 0)),
        pl.BlockSpec((1, C1), lambda i: (0, 0)),
        pl.BlockSpec((9 * C1, C2), lambda i: (0, 0)),
        pl.BlockSpec((1, C2), lambda i: (0, 0)),
        pl.BlockSpec((1, C2), lambda i: (0, 0)),
    ]
    args = [x, w1.astype(_CDT), _row(s1, C1), _row(t1, C1),
            w2.astype(_CDT), _row(s2, C2), _row(t2, C2)]
    if use1x1:
        in_specs += [pl.BlockSpec((sc_hi - sc_lo, C2), lambda i: (0, 0)),
                     pl.BlockSpec((1, C2), lambda i: (0, 0))]
        args += [ws.astype(_CDT), _row(tb, C2)]
    if fuse_head:
        hw1, hb1, hw2, hb2 = head
        hid = hw1.shape[1]
        in_specs += [pl.BlockSpec((C2, hid), lambda i: (0, 0)),
                     pl.BlockSpec((1, hid), lambda i: (0, 0)),
                     pl.BlockSpec((1, hid), lambda i: (0, 0)),
                     pl.BlockSpec((1, 1), lambda i: (0, 0))]
        args += [hw1.astype(jnp.float32), _row(hb1, hid), _row(hw2, hid),
                 hb2.reshape(1, 1).astype(jnp.float32)]
        out_shape = jax.ShapeDtypeStruct((B, 1), jnp.float32)
        out_spec = pl.BlockSpec((bblk, 1), lambda i: (i, 0))
    elif pad_out:
        out_shape = jax.ShapeDtypeStruct((B, Ho + 2, Wo + 2, C2), _CDT)
        out_spec = pl.BlockSpec((bblk, Ho + 2, Wo + 2, C2),
                                lambda i: (i, 0, 0, 0))
    else:
        out_shape = jax.ShapeDtypeStruct((B, Ho, Wo, C2), _CDT)
        out_spec = pl.BlockSpec((bblk, Ho, Wo, C2), lambda i: (i, 0, 0, 0))

    body = functools.partial(
        _unit_body, taps=taps, Ho=Ho, Wo=Wo, C1=C1, C2=C2, use1x1=use1x1,
        sc_h0=sc_h0, sc_lo=sc_lo, sc_hi=sc_hi, pad_out=pad_out,
        fuse_head=fuse_head)
    return pl.pallas_call(
        body,
        grid=grid,
        in_specs=in_specs,
        out_specs=out_spec,
        out_shape=out_shape,
        scratch_shapes=[pltpu.VMEM((bblk, Ho + 2, Wo + 2, C1), _CDT)],
        compiler_params=pltpu.CompilerParams(
            dimension_semantics=("parallel",)),
    )(*args)


def _s2d_weight(w, cin, cout, pad_to=None):
    w4 = jnp.zeros((2, 2, 2, 2, cin, cout), w.dtype)
    mp = {0: (0, 0), 1: (0, 1), 2: (1, 0)}
    for dh in range(3):
        di, pr = mp[dh]
        for dw in range(3):
            dj, pc = mp[dw]
            w4 = w4.at[di, dj, pr, pc].set(w[dh, dw])
    w4 = w4.reshape(2, 2, 4 * cin, cout)
    if pad_to is not None and pad_to > 4 * cin:
        w4 = jnp.pad(w4, ((0, 0), (0, 0), (0, pad_to - 4 * cin), (0, 0)))
    return w4.reshape(-1, cout)


def _s2d(x_pad):
    B, H2, W2, C = x_pad.shape
    x = x_pad.reshape(B, H2 // 2, 2, W2 // 2, 2, C)
    x = jnp.transpose(x, (0, 1, 3, 2, 4, 5))
    return x.reshape(B, H2 // 2, W2 // 2, 4 * C)


def kernel(x, mbh, b0u0_conv1_w, b0u0_conv1_b, b0u0_conv2_w, b0u0_conv2_b, b0u0_bn1_g, b0u0_bn1_be, b0u0_bn1_m, b0u0_bn1_v, b0u0_bn2_g, b0u0_bn2_be, b0u0_bn2_m, b0u0_bn2_v, b0u0_conv3_w, b0u0_conv3_b, b0u1_conv1_w, b0u1_conv1_b, b0u1_conv2_w, b0u1_conv2_b, b0u1_bn1_g, b0u1_bn1_be, b0u1_bn1_m, b0u1_bn1_v, b0u1_bn2_g, b0u1_bn2_be, b0u1_bn2_m, b0u1_bn2_v, b1u0_conv1_w, b1u0_conv1_b, b1u0_conv2_w, b1u0_conv2_b, b1u0_bn1_g, b1u0_bn1_be, b1u0_bn1_m, b1u0_bn1_v, b1u0_bn2_g, b1u0_bn2_be, b1u0_bn2_m, b1u0_bn2_v, b1u0_conv3_w, b1u0_conv3_b, b1u1_conv1_w, b1u1_conv1_b, b1u1_conv2_w, b1u1_conv2_b, b1u1_bn1_g, b1u1_bn1_be, b1u1_bn1_m, b1u1_bn1_v, b1u1_bn2_g, b1u1_bn2_be, b1u1_bn2_m, b1u1_bn2_v, b2u0_conv1_w, b2u0_conv1_b, b2u0_conv2_w, b2u0_conv2_b, b2u0_bn1_g, b2u0_bn1_be, b2u0_bn1_m, b2u0_bn1_v, b2u0_bn2_g, b2u0_bn2_be, b2u0_bn2_m, b2u0_bn2_v, b2u0_conv3_w, b2u0_conv3_b, b2u1_conv1_w, b2u1_conv1_b, b2u1_conv2_w, b2u1_conv2_b, b2u1_bn1_g, b2u1_bn1_be, b2u1_bn1_m, b2u1_bn1_v, b2u1_bn2_g, b2u1_bn2_be, b2u1_bn2_m, b2u1_bn2_v, b3u0_conv1_w, b3u0_conv1_b, b3u0_conv2_w, b3u0_conv2_b, b3u0_bn1_g, b3u0_bn1_be, b3u0_bn1_m, b3u0_bn1_v, b3u0_bn2_g, b3u0_bn2_be, b3u0_bn2_m, b3u0_bn2_v, b3u0_conv3_w, b3u0_conv3_b, b3u1_conv1_w, b3u1_conv1_b, b3u1_conv2_w, b3u1_conv2_b, b3u1_bn1_g, b3u1_bn1_be, b3u1_bn1_m, b3u1_bn1_v, b3u1_bn2_g, b3u1_bn2_be, b3u1_bn2_m, b3u1_bn2_v, w1_feat, w1_mbh, b1, w2, b2):
    B = x.shape[0]

    xt = jnp.transpose(x, (0, 2, 3, 1))
    xp = jnp.pad(xt, ((0, 0), (1, 1), (1, 1), (0, 0)))
    x0 = _s2d(xp)
    x0 = jnp.pad(x0, ((0, 0), (0, 0), (0, 0), (0, 4))).astype(_CDT)

    s1, t1 = _fold_bn(b0u0_conv1_b, b0u0_bn1_g, b0u0_bn1_be, b0u0_bn1_m,
                      b0u0_bn1_v)
    s2, t2 = _fold_bn(b0u0_conv2_b, b0u0_bn2_g, b0u0_bn2_be, b0u0_bn2_m,
                      b0u0_bn2_v)
    w1 = _s2d_weight(b0u0_conv1_w, 15, 32, pad_to=64)
    y = _res_unit(x0, w1, s1, t1, b0u0_conv2_w.reshape(-1, 32), s2, t2,
                  ws=b0u0_conv3_w.reshape(15, 32), tb=b0u0_conv3_b,
                  taps=_TAPS4, Ho=40, Wo=40, sc_h0=0, sc_lo=45, sc_hi=60,
                  pad_out=True)

    s1, t1 = _fold_bn(b0u1_conv1_b, b0u1_bn1_g, b0u1_bn1_be, b0u1_bn1_m,
                      b0u1_bn1_v)
    s2, t2 = _fold_bn(b0u1_conv2_b, b0u1_bn2_g, b0u1_bn2_be, b0u1_bn2_m,
                      b0u1_bn2_v)
    y = _res_unit(y, b0u1_conv1_w.reshape(-1, 32), s1, t1,
                  b0u1_conv2_w.reshape(-1, 32), s2, t2,
                  taps=_TAPS9, Ho=40, Wo=40, sc_h0=1, sc_lo=0, sc_hi=32,
                  pad_out=True)

    y = _s2d(y)
    s1, t1 = _fold_bn(b1u0_conv1_b, b1u0_bn1_g, b1u0_bn1_be, b1u0_bn1_m,
                      b1u0_bn1_v)
    s2, t2 = _fold_bn(b1u0_conv2_b, b1u0_bn2_g, b1u0_bn2_be, b1u0_bn2_m,
                      b1u0_bn2_v)
    w1 = _s2d_weight(b1u0_conv1_w, 32, 64)
    y = _res_unit(y, w1, s1, t1, b1u0_conv2_w.reshape(-1, 64), s2, t2,
                  ws=b1u0_conv3_w.reshape(32, 64), tb=b1u0_conv3_b,
                  taps=_TAPS4, Ho=20, Wo=20, sc_h0=0, sc_lo=96, sc_hi=128,
                  pad_out=True)

    s1, t1 = _fold_bn(b1u1_conv1_b, b1u1_bn1_g, b1u1_bn1_be, b1u1_bn1_m,
                      b1u1_bn1_v)
    s2, t2 = _fold_bn(b1u1_conv2_b, b1u1_bn2_g, b1u1_bn2_be, b1u1_bn2_m,
                      b1u1_bn2_v)
    y = _res_unit(y, b1u1_conv1_w.reshape(-1, 64), s1, t1,
                  b1u1_conv2_w.reshape(-1, 64), s2, t2,
                  taps=_TAPS9, Ho=20, Wo=20, sc_h0=1, sc_lo=0, sc_hi=64,
                  pad_out=True)

    s1, t1 = _fold_bn(b2u0_conv1_b, b2u0_bn1_g, b2u0_bn1_be, b2u0_bn1_m,
                      b2u0_bn1_v)
    s2, t2 = _fold_bn(b2u0_conv2_b, b2u0_bn2_g, b2u0_bn2_be, b2u0_bn2_m,
                      b2u0_bn2_v)
    y = _res_unit(y, b2u0_conv1_w.reshape(-1, 128), s1, t1,
                  b2u0_conv2_w.reshape(-1, 128), s2, t2,
                  ws=b2u0_conv3_w.reshape(64, 128), tb=b2u0_conv3_b,
                  taps=_TAPS9, Ho=20, Wo=20, sc_h0=1, sc_lo=0, sc_hi=64,
                  pad_out=True)

    s1, t1 = _fold_bn(b2u1_conv1_b, b2u1_bn1_g, b2u1_bn1_be, b2u1_bn1_m,
                      b2u1_bn1_v)
    s2, t2 = _fold_bn(b2u1_conv2_b, b2u1_bn2_g, b2u1_bn2_be, b2u1_bn2_m,
                      b2u1_bn2_v)
    y = _res_unit(y, b2u1_conv1_w.reshape(-1, 128), s1, t1,
                  b2u1_conv2_w.reshape(-1, 128), s2, t2,
                  taps=_TAPS9, Ho=20, Wo=20, sc_h0=1, sc_lo=0, sc_hi=128,
                  pad_out=True)

    s1, t1 = _fold_bn(b3u0_conv1_b, b3u0_bn1_g, b3u0_bn1_be, b3u0_bn1_m,
                      b3u0_bn1_v)
    s2, t2 = _fold_bn(b3u0_conv2_b, b3u0_bn2_g, b3u0_bn2_be, b3u0_bn2_m,
                      b3u0_bn2_v)
    y = _res_unit(y, b3u0_conv1_w.reshape(-1, 256), s1, t1,
                  b3u0_conv2_w.reshape(-1, 256), s2, t2,
                  ws=b3u0_conv3_w.reshape(128, 256), tb=b3u0_conv3_b,
                  taps=_TAPS9, Ho=20, Wo=20, sc_h0=1, sc_lo=0, sc_hi=128,
                  pad_out=True)

    s1, t1 = _fold_bn(b3u1_conv1_b, b3u1_bn1_g, b3u1_bn1_be, b3u1_bn1_m,
                      b3u1_bn1_v)
    s2, t2 = _fold_bn(b3u1_conv2_b, b3u1_bn2_g, b3u1_bn2_be, b3u1_bn2_m,
                      b3u1_bn2_v)
    hb1 = b1.reshape(1, -1) + jnp.asarray(mbh, jnp.float32).reshape(1, 1) * \
        w1_mbh.reshape(1, -1)
    out = _res_unit(y, b3u1_conv1_w.reshape(-1, 256), s1, t1,
                    b3u1_conv2_w.reshape(-1, 256), s2, t2,
                    head=(w1_feat, hb1, w2.reshape(1, -1), b2),
                    taps=_TAPS9, Ho=20, Wo=20, sc_h0=1, sc_lo=0, sc_hi=256,
                    pad_out=False)
    return out
